# Initial kernel scaffold; baseline (speedup 1.0000x reference)
import functools

import jax
import jax.numpy as jnp
from jax import lax
from jax.experimental import pallas as pl
from jax.experimental.pallas import tpu as pltpu
from jax.experimental.pallas import tpu_sc as plsc


def _edge_mlp_body(rbf_ref, w1_ref, b1_ref, w2_ref, b2_ref, out_ref):
    h = jnp.dot(rbf_ref[...], w1_ref[...], preferred_element_type=jnp.float32)
    h = h + b1_ref[...]
    h = h * jax.nn.sigmoid(h)
    o = jnp.dot(h, w2_ref[...], preferred_element_type=jnp.float32)
    out_ref[...] = o + b2_ref[...]


def _edge_mlp(rbf, w1, b1, w2, b2, block_e):
    e, r = rbf.shape
    h = w1.shape[1]
    grid = e // block_e
    return pl.pallas_call(
        _edge_mlp_body,
        grid=(grid,),
        in_specs=[
            pl.BlockSpec((block_e, r), lambda i: (i, 0)),
            pl.BlockSpec((r, h), lambda i: (0, 0)),
            pl.BlockSpec((1, h), lambda i: (0, 0)),
            pl.BlockSpec((h, h), lambda i: (0, 0)),
            pl.BlockSpec((1, h), lambda i: (0, 0)),
        ],
        out_specs=pl.BlockSpec((block_e, h), lambda i: (i, 0)),
        out_shape=jax.ShapeDtypeStruct((e, h), jnp.float32),
    )(rbf, w1, b1, w2, b2)


_K = 128
_UNROLL = 4


def _sc_aggregate(ef, x, row, col, zeros):
    e, h = ef.shape
    n = x.shape[0]
    nchunks = e // _K
    nw = 32
    tpw = -(-nchunks // nw)
    rows_per_tile = n // 16

    mesh = plsc.VectorSubcoreMesh(core_axis_name="c", subcore_axis_name="s")

    @functools.partial(
        pl.kernel,
        mesh=mesh,
        out_type=jax.ShapeDtypeStruct((2, n, h), jnp.float32),
        scratch_types=[
            pltpu.VMEM((_K,), jnp.int32),
            pltpu.VMEM((_K,), jnp.int32),
            pltpu.VMEM((_K, h), jnp.float32),
            pltpu.VMEM((_K, h), jnp.float32),
            pltpu.VMEM((_K, h), jnp.float32),
            pltpu.VMEM_SHARED((n, h), jnp.float32),
            pltpu.SemaphoreType.DMA,
        ],
    )
    def sc_kernel(ef_hbm, x_hbm, row_hbm, col_hbm, zero_hbm, out_hbm,
                  col_v, row_v, gx_v, ef_v, msg_v, agg_sh, sem):
        cid = lax.axis_index("c")
        sid = lax.axis_index("s")
        wid = sid * 2 + cid

        zbase = sid * rows_per_tile
        pltpu.sync_copy(zero_hbm.at[pl.ds(zbase, rows_per_tile)],
                        agg_sh.at[pl.ds(zbase, rows_per_tile)])
        plsc.subcore_barrier()

        def chunk_body(t, carry):
            chunk = t * nw + wid

            @pl.when(chunk < nchunks)
            def _():
                base = chunk * _K
                pltpu.sync_copy(col_hbm.at[pl.ds(base, _K)], col_v)
                pltpu.sync_copy(row_hbm.at[pl.ds(base, _K)], row_v)
                gcp = pltpu.async_copy(x_hbm.at[col_v], gx_v, sem)
                pltpu.sync_copy(ef_hbm.at[pl.ds(base, _K)], ef_v)
                gcp.wait()

                def mul_body(i, c):
                    for u in range(_UNROLL):
                        ei = i * _UNROLL + u
                        for j in range(h // 16):
                            sl = pl.ds(j * 16, 16)
                            msg_v[ei, sl] = ef_v[ei, sl] * gx_v[ei, sl]
                    return c

                lax.fori_loop(0, _K // _UNROLL, mul_body, 0, unroll=False)
                pltpu.sync_copy(msg_v, agg_sh.at[row_v], add=True)

            return carry

        lax.fori_loop(0, tpw, chunk_body, 0, unroll=False)

        plsc.subcore_barrier()
        pltpu.sync_copy(agg_sh.at[pl.ds(zbase, rows_per_tile)],
                        out_hbm.at[cid, pl.ds(zbase, rows_per_tile)])

    return sc_kernel(ef, x, row, col, zeros)


def _node_mlp_body(x_ref, p0_ref, p1_ref, w1_ref, b1_ref, w2_ref, b2_ref,
                   out_ref):
    agg = p0_ref[...] + p1_ref[...]
    g = jnp.dot(agg, w1_ref[...], preferred_element_type=jnp.float32)
    g = g + b1_ref[...]
    g = g * jax.nn.sigmoid(g)
    o = jnp.dot(g, w2_ref[...], preferred_element_type=jnp.float32)
    out_ref[...] = x_ref[...] + o + b2_ref[...]


def _node_mlp(x, p0, p1, w1, b1, w2, b2, block_n):
    n, h = x.shape
    grid = n // block_n
    return pl.pallas_call(
        _node_mlp_body,
        grid=(grid,),
        in_specs=[
            pl.BlockSpec((block_n, h), lambda i: (i, 0)),
            pl.BlockSpec((block_n, h), lambda i: (i, 0)),
            pl.BlockSpec((block_n, h), lambda i: (i, 0)),
            pl.BlockSpec((h, h), lambda i: (0, 0)),
            pl.BlockSpec((1, h), lambda i: (0, 0)),
            pl.BlockSpec((h, h), lambda i: (0, 0)),
            pl.BlockSpec((1, h), lambda i: (0, 0)),
        ],
        out_specs=pl.BlockSpec((block_n, h), lambda i: (i, 0)),
        out_shape=jax.ShapeDtypeStruct((n, h), jnp.float32),
    )(x, p0, p1, w1, b1, w2, b2)


def kernel(x, edge_index, edge_rbf, We1, be1, We2, be2, Wn1, bn1, Wn2, bn2):
    n, h = x.shape
    e = edge_rbf.shape[0]
    assert e % _K == 0 and n % 16 == 0 and h % 16 == 0

    row = edge_index[0]
    col = edge_index[1]

    ef = _edge_mlp(edge_rbf, We1, be1.reshape(1, h), We2, be2.reshape(1, h),
                   block_e=3200)

    zeros = jnp.zeros((n, h), jnp.float32)
    partials = _sc_aggregate(ef, x, row, col, zeros)

    return _node_mlp(x, partials[0], partials[1], Wn1, bn1.reshape(1, h),
                     Wn2, bn2.reshape(1, h), block_n=2000)


# speedup vs baseline: 3.3044x; 3.3044x over previous
import functools

import jax
import jax.numpy as jnp
from jax import lax
from jax.experimental import pallas as pl
from jax.experimental.pallas import tpu as pltpu
from jax.experimental.pallas import tpu_sc as plsc


def _edge_mlp_body(rbf_ref, w1_ref, b1_ref, w2_ref, b2_ref, out_ref):
    h = jnp.dot(rbf_ref[...], w1_ref[...], preferred_element_type=jnp.float32)
    h = h + b1_ref[...]
    h = h * jax.nn.sigmoid(h)
    o = jnp.dot(h, w2_ref[...], preferred_element_type=jnp.float32)
    out_ref[...] = o + b2_ref[...]


def _edge_mlp(rbf, w1, b1, w2, b2, block_e):
    e, r = rbf.shape
    h = w1.shape[1]
    grid = e // block_e
    return pl.pallas_call(
        _edge_mlp_body,
        grid=(grid,),
        in_specs=[
            pl.BlockSpec((block_e, r), lambda i: (i, 0)),
            pl.BlockSpec((r, h), lambda i: (0, 0)),
            pl.BlockSpec((1, h), lambda i: (0, 0)),
            pl.BlockSpec((h, h), lambda i: (0, 0)),
            pl.BlockSpec((1, h), lambda i: (0, 0)),
        ],
        out_specs=pl.BlockSpec((block_e, h), lambda i: (i, 0)),
        out_shape=jax.ShapeDtypeStruct((e, h), jnp.float32),
    )(rbf, w1, b1, w2, b2)


_K = 128
_UNROLL = 4


def _sc_aggregate(ef, x, row, col, zeros):
    e, h = ef.shape
    n = x.shape[0]
    nchunks = e // _K
    nw = 32
    tpw = -(-nchunks // nw)
    rows_per_tile = (n // 128) * 8
    tail_base = rows_per_tile * 16
    tail_rows = n - tail_base

    mesh = plsc.VectorSubcoreMesh(core_axis_name="c", subcore_axis_name="s")

    @functools.partial(
        pl.kernel,
        mesh=mesh,
        out_type=jax.ShapeDtypeStruct((2, n, h), jnp.float32),
        scratch_types=[
            pltpu.VMEM((_K,), jnp.int32),
            pltpu.VMEM((_K,), jnp.int32),
            pltpu.VMEM((_K, h), jnp.float32),
            pltpu.VMEM((_K, h), jnp.float32),
            pltpu.VMEM((_K, h), jnp.float32),
            pltpu.VMEM_SHARED((n, h), jnp.float32),
            pltpu.SemaphoreType.DMA,
        ],
    )
    def sc_kernel(ef_hbm, x_hbm, row_hbm, col_hbm, zero_hbm, out_hbm,
                  col_v, row_v, gx_v, ef_v, msg_v, agg_sh, sem):
        cid = lax.axis_index("c")
        sid = lax.axis_index("s")
        wid = sid * 2 + cid

        zbase = sid * rows_per_tile
        pltpu.sync_copy(zero_hbm.at[pl.ds(zbase, rows_per_tile)],
                        agg_sh.at[pl.ds(zbase, rows_per_tile)])
        if tail_rows:
            @pl.when(sid == 0)
            def _zero_tail():
                pltpu.sync_copy(zero_hbm.at[pl.ds(tail_base, tail_rows)],
                                agg_sh.at[pl.ds(tail_base, tail_rows)])
        plsc.subcore_barrier()

        def chunk_body(t, carry):
            chunk = t * nw + wid

            @pl.when(chunk < nchunks)
            def _():
                base = chunk * _K
                pltpu.sync_copy(col_hbm.at[pl.ds(base, _K)], col_v)
                pltpu.sync_copy(row_hbm.at[pl.ds(base, _K)], row_v)
                gcp = pltpu.async_copy(x_hbm.at[col_v], gx_v, sem)
                pltpu.sync_copy(ef_hbm.at[pl.ds(base, _K)], ef_v)
                gcp.wait()

                def mul_body(i, c):
                    for u in range(_UNROLL):
                        ei = i * _UNROLL + u
                        for j in range(h // 16):
                            sl = pl.ds(j * 16, 16)
                            msg_v[ei, sl] = ef_v[ei, sl] * gx_v[ei, sl]
                    return c

                lax.fori_loop(0, _K // _UNROLL, mul_body, 0, unroll=False)
                pltpu.sync_copy(msg_v, agg_sh.at[row_v], add=True)

            return carry

        lax.fori_loop(0, tpw, chunk_body, 0, unroll=False)

        plsc.subcore_barrier()
        pltpu.sync_copy(agg_sh.at[pl.ds(zbase, rows_per_tile)],
                        out_hbm.at[cid, pl.ds(zbase, rows_per_tile)])
        if tail_rows:
            @pl.when(sid == 0)
            def _write_tail():
                pltpu.sync_copy(agg_sh.at[pl.ds(tail_base, tail_rows)],
                                out_hbm.at[cid, pl.ds(tail_base, tail_rows)])

    return sc_kernel(ef, x, row, col, zeros)


def _node_mlp_body(x_ref, p0_ref, p1_ref, w1_ref, b1_ref, w2_ref, b2_ref,
                   out_ref):
    agg = p0_ref[...] + p1_ref[...]
    g = jnp.dot(agg, w1_ref[...], preferred_element_type=jnp.float32)
    g = g + b1_ref[...]
    g = g * jax.nn.sigmoid(g)
    o = jnp.dot(g, w2_ref[...], preferred_element_type=jnp.float32)
    out_ref[...] = x_ref[...] + o + b2_ref[...]


def _node_mlp(x, p0, p1, w1, b1, w2, b2, block_n):
    n, h = x.shape
    grid = n // block_n
    return pl.pallas_call(
        _node_mlp_body,
        grid=(grid,),
        in_specs=[
            pl.BlockSpec((block_n, h), lambda i: (i, 0)),
            pl.BlockSpec((block_n, h), lambda i: (i, 0)),
            pl.BlockSpec((block_n, h), lambda i: (i, 0)),
            pl.BlockSpec((h, h), lambda i: (0, 0)),
            pl.BlockSpec((1, h), lambda i: (0, 0)),
            pl.BlockSpec((h, h), lambda i: (0, 0)),
            pl.BlockSpec((1, h), lambda i: (0, 0)),
        ],
        out_specs=pl.BlockSpec((block_n, h), lambda i: (i, 0)),
        out_shape=jax.ShapeDtypeStruct((n, h), jnp.float32),
    )(x, p0, p1, w1, b1, w2, b2)


def kernel(x, edge_index, edge_rbf, We1, be1, We2, be2, Wn1, bn1, Wn2, bn2):
    n, h = x.shape
    e = edge_rbf.shape[0]
    assert e % _K == 0 and n % 16 == 0 and h % 16 == 0

    row = edge_index[0]
    col = edge_index[1]

    ef = _edge_mlp(edge_rbf, We1, be1.reshape(1, h), We2, be2.reshape(1, h),
                   block_e=3200)

    zeros = jnp.zeros((n, h), jnp.float32)
    partials = _sc_aggregate(ef, x, row, col, zeros)

    return _node_mlp(x, partials[0], partials[1], Wn1, bn1.reshape(1, h),
                     Wn2, bn2.reshape(1, h), block_n=2000)


# speedup vs baseline: 4.9234x; 1.4899x over previous
import functools

import jax
import jax.numpy as jnp
from jax import lax
from jax.experimental import pallas as pl
from jax.experimental.pallas import tpu as pltpu
from jax.experimental.pallas import tpu_sc as plsc


def _edge_mlp_body(rbf_ref, w1_ref, b1_ref, w2_ref, b2_ref, out_ref):
    h = jnp.dot(rbf_ref[...], w1_ref[...], preferred_element_type=jnp.float32)
    h = h + b1_ref[...]
    h = h * jax.nn.sigmoid(h)
    o = jnp.dot(h, w2_ref[...], preferred_element_type=jnp.float32)
    out_ref[...] = o + b2_ref[...]


def _edge_mlp(rbf, w1, b1, w2, b2, block_e):
    e, r = rbf.shape
    h = w1.shape[1]
    grid = e // block_e
    return pl.pallas_call(
        _edge_mlp_body,
        grid=(grid,),
        in_specs=[
            pl.BlockSpec((block_e, r), lambda i: (i, 0)),
            pl.BlockSpec((r, h), lambda i: (0, 0)),
            pl.BlockSpec((1, h), lambda i: (0, 0)),
            pl.BlockSpec((h, h), lambda i: (0, 0)),
            pl.BlockSpec((1, h), lambda i: (0, 0)),
        ],
        out_specs=pl.BlockSpec((block_e, h), lambda i: (i, 0)),
        out_shape=jax.ShapeDtypeStruct((e, h), jnp.float32),
    )(rbf, w1, b1, w2, b2)


_K = 64


def _sc_aggregate(ef, x, row, col, zeros):
    e, h = ef.shape
    n = x.shape[0]
    nchunks = e // _K
    nw = 32
    tpw = -(-nchunks // nw)
    rows_per_tile = (n // 128) * 8
    tail_base = rows_per_tile * 16
    tail_rows = n - tail_base

    mesh = plsc.VectorSubcoreMesh(core_axis_name="c", subcore_axis_name="s")

    @functools.partial(
        pl.kernel,
        mesh=mesh,
        out_type=jax.ShapeDtypeStruct((2, n, h), jnp.float32),
        scratch_types=[
            pltpu.VMEM((4, _K), jnp.int32),
            pltpu.VMEM((4, _K), jnp.int32),
            pltpu.VMEM((_K, h), jnp.float32),
            pltpu.VMEM((_K, h), jnp.float32),
            pltpu.VMEM((_K, h), jnp.float32),
            pltpu.VMEM((_K, h), jnp.float32),
            pltpu.VMEM((_K, h), jnp.float32),
            pltpu.VMEM((_K, h), jnp.float32),
            pltpu.VMEM_SHARED((n, h), jnp.float32),
            [pltpu.SemaphoreType.DMA] * 4,
            [pltpu.SemaphoreType.DMA] * 4,
            [pltpu.SemaphoreType.DMA] * 2,
            [pltpu.SemaphoreType.DMA] * 2,
            [pltpu.SemaphoreType.DMA] * 2,
        ],
    )
    def sc_kernel(ef_hbm, x_hbm, row_hbm, col_hbm, zero_hbm, out_hbm,
                  col_v, row_v, gx0, gx1, ef0, ef1, msg0, msg1, agg_sh,
                  sem_col, sem_row, sem_ef, sem_gx, sem_sc):
        gx_v = (gx0, gx1)
        ef_v = (ef0, ef1)
        msg_v = (msg0, msg1)
        cid = lax.axis_index("c")
        sid = lax.axis_index("s")
        wid = sid * 2 + cid

        zbase = sid * rows_per_tile
        pltpu.sync_copy(zero_hbm.at[pl.ds(zbase, rows_per_tile)],
                        agg_sh.at[pl.ds(zbase, rows_per_tile)])
        if tail_rows:
            @pl.when(sid == 0)
            def _zero_tail():
                pltpu.sync_copy(zero_hbm.at[pl.ds(tail_base, tail_rows)],
                                agg_sh.at[pl.ds(tail_base, tail_rows)])
        plsc.subcore_barrier()

        def load_start(t, u4, b2):
            base = (t * nw + wid) * _K
            pltpu.async_copy(col_hbm.at[pl.ds(base, _K)], col_v.at[u4],
                             sem_col[u4])
            pltpu.async_copy(row_hbm.at[pl.ds(base, _K)], row_v.at[u4],
                             sem_row[u4])
            pltpu.async_copy(ef_hbm.at[pl.ds(base, _K)], ef_v[b2],
                             sem_ef[b2])

        def gather_start(u4, b2):
            pltpu.make_async_copy(col_hbm.at[pl.ds(0, _K)], col_v.at[u4],
                                  sem_col[u4]).wait()
            pltpu.async_copy(x_hbm.at[col_v.at[u4]], gx_v[b2], sem_gx[b2])

        load_start(0, 0, 0)
        load_start(1, 1, 1)
        gather_start(0, 0)

        def quad_body(q, carry):
            for u in range(4):
                b2 = u % 2
                nb2 = (u + 1) % 2
                t = q * 4 + u
                chunk = t * nw + wid

                @pl.when(chunk + nw < nchunks)
                def _():
                    gather_start((u + 1) % 4, nb2)

                @pl.when(chunk < nchunks)
                def _():
                    pltpu.make_async_copy(ef_hbm.at[pl.ds(0, _K)], ef_v[b2],
                                          sem_ef[b2]).wait()
                    pltpu.make_async_copy(x_hbm.at[col_v.at[u]], gx_v[b2],
                                          sem_gx[b2]).wait()

                    @pl.when(t >= 2)
                    def _():
                        pltpu.make_async_copy(
                            msg_v[b2], agg_sh.at[row_v.at[u]],
                            sem_sc[b2]).wait()

                    @plsc.parallel_loop(0, _K, unroll=8)
                    def _mul(i):
                        for j in range(h // 16):
                            sl = pl.ds(j * 16, 16)
                            msg_v[b2][i, sl] = ef_v[b2][i, sl] * gx_v[b2][i, sl]

                    pltpu.make_async_copy(row_hbm.at[pl.ds(0, _K)],
                                          row_v.at[u], sem_row[u]).wait()
                    pltpu.async_copy(msg_v[b2], agg_sh.at[row_v.at[u]],
                                     sem_sc[b2], add=True)

                    @pl.when(chunk + 2 * nw >= nchunks)
                    def _():
                        pltpu.make_async_copy(
                            msg_v[b2], agg_sh.at[row_v.at[u]],
                            sem_sc[b2]).wait()

                @pl.when(chunk + 2 * nw < nchunks)
                def _():
                    load_start(t + 2, (u + 2) % 4, b2)

            return carry

        lax.fori_loop(0, -(-tpw // 4), quad_body, 0, unroll=False)

        plsc.subcore_barrier()
        pltpu.sync_copy(agg_sh.at[pl.ds(zbase, rows_per_tile)],
                        out_hbm.at[cid, pl.ds(zbase, rows_per_tile)])
        if tail_rows:
            @pl.when(sid == 0)
            def _write_tail():
                pltpu.sync_copy(agg_sh.at[pl.ds(tail_base, tail_rows)],
                                out_hbm.at[cid, pl.ds(tail_base, tail_rows)])

    return sc_kernel(ef, x, row, col, zeros)


def _node_mlp_body(x_ref, p0_ref, p1_ref, w1_ref, b1_ref, w2_ref, b2_ref,
                   out_ref):
    agg = p0_ref[...] + p1_ref[...]
    g = jnp.dot(agg, w1_ref[...], preferred_element_type=jnp.float32)
    g = g + b1_ref[...]
    g = g * jax.nn.sigmoid(g)
    o = jnp.dot(g, w2_ref[...], preferred_element_type=jnp.float32)
    out_ref[...] = x_ref[...] + o + b2_ref[...]


def _node_mlp(x, p0, p1, w1, b1, w2, b2, block_n):
    n, h = x.shape
    grid = n // block_n
    return pl.pallas_call(
        _node_mlp_body,
        grid=(grid,),
        in_specs=[
            pl.BlockSpec((block_n, h), lambda i: (i, 0)),
            pl.BlockSpec((block_n, h), lambda i: (i, 0)),
            pl.BlockSpec((block_n, h), lambda i: (i, 0)),
            pl.BlockSpec((h, h), lambda i: (0, 0)),
            pl.BlockSpec((1, h), lambda i: (0, 0)),
            pl.BlockSpec((h, h), lambda i: (0, 0)),
            pl.BlockSpec((1, h), lambda i: (0, 0)),
        ],
        out_specs=pl.BlockSpec((block_n, h), lambda i: (i, 0)),
        out_shape=jax.ShapeDtypeStruct((n, h), jnp.float32),
    )(x, p0, p1, w1, b1, w2, b2)


def kernel(x, edge_index, edge_rbf, We1, be1, We2, be2, Wn1, bn1, Wn2, bn2):
    n, h = x.shape
    e = edge_rbf.shape[0]
    assert e % _K == 0 and n % 16 == 0 and h % 16 == 0

    row = edge_index[0]
    col = edge_index[1]

    ef = _edge_mlp(edge_rbf, We1, be1.reshape(1, h), We2, be2.reshape(1, h),
                   block_e=3200)

    zeros = jnp.zeros((n, h), jnp.float32)
    partials = _sc_aggregate(ef, x, row, col, zeros)

    return _node_mlp(x, partials[0], partials[1], Wn1, bn1.reshape(1, h),
                     Wn2, bn2.reshape(1, h), block_n=2000)


# speedup vs baseline: 5.1968x; 1.0555x over previous
import functools

import jax
import jax.numpy as jnp
from jax import lax
from jax.experimental import pallas as pl
from jax.experimental.pallas import tpu as pltpu
from jax.experimental.pallas import tpu_sc as plsc


def _edge_mlp_body(rbf_ref, w1_ref, b1_ref, w2_ref, b2_ref, out_ref):
    h = jnp.dot(rbf_ref[...], w1_ref[...], preferred_element_type=jnp.float32)
    h = h + b1_ref[...]
    h = h * jax.nn.sigmoid(h)
    o = jnp.dot(h, w2_ref[...], preferred_element_type=jnp.float32)
    ob = (o + b2_ref[...]).astype(jnp.bfloat16)
    out_ref[...] = pltpu.bitcast(ob, jnp.uint32)


def _edge_mlp(rbf, w1, b1, w2, b2, block_e):
    e, r = rbf.shape
    h = w1.shape[1]
    grid = e // block_e
    return pl.pallas_call(
        _edge_mlp_body,
        grid=(grid,),
        in_specs=[
            pl.BlockSpec((block_e, r), lambda i: (i, 0)),
            pl.BlockSpec((r, h), lambda i: (0, 0)),
            pl.BlockSpec((1, h), lambda i: (0, 0)),
            pl.BlockSpec((h, h), lambda i: (0, 0)),
            pl.BlockSpec((1, h), lambda i: (0, 0)),
        ],
        out_specs=pl.BlockSpec((block_e // 2, h), lambda i: (i, 0)),
        out_shape=jax.ShapeDtypeStruct((e // 2, h), jnp.uint32),
    )(rbf, w1, b1, w2, b2)


_K = 64


def _sc_aggregate(ef, x, row, col, zeros):
    e2 = ef.shape[0]
    e = e2 * 2
    n, h = x.shape
    nchunks = e // _K
    nw = 32
    tpw = -(-nchunks // nw)
    rows_per_tile = (n // 128) * 8
    tail_base = rows_per_tile * 16
    tail_rows = n - tail_base

    mesh = plsc.VectorSubcoreMesh(core_axis_name="c", subcore_axis_name="s")

    @functools.partial(
        pl.kernel,
        mesh=mesh,
        out_type=jax.ShapeDtypeStruct((2, n, h), jnp.float32),
        scratch_types=[
            pltpu.VMEM((4, _K), jnp.int32),
            pltpu.VMEM((4, _K), jnp.int32),
            pltpu.VMEM((_K, h), jnp.float32),
            pltpu.VMEM((_K, h), jnp.float32),
            pltpu.VMEM((_K // 2, h), jnp.uint32),
            pltpu.VMEM((_K // 2, h), jnp.uint32),
            pltpu.VMEM((_K, h), jnp.float32),
            pltpu.VMEM((_K, h), jnp.float32),
            pltpu.VMEM_SHARED((n, h), jnp.float32),
            [pltpu.SemaphoreType.DMA] * 4,
            [pltpu.SemaphoreType.DMA] * 4,
            [pltpu.SemaphoreType.DMA] * 2,
            [pltpu.SemaphoreType.DMA] * 2,
            [pltpu.SemaphoreType.DMA] * 2,
        ],
    )
    def sc_kernel(ef_hbm, x_hbm, row_hbm, col_hbm, zero_hbm, out_hbm,
                  col_v, row_v, gx0, gx1, ef0, ef1, msg0, msg1, agg_sh,
                  sem_col, sem_row, sem_ef, sem_gx, sem_sc):
        gx_v = (gx0, gx1)
        ef_v = (ef0, ef1)
        msg_v = (msg0, msg1)
        cid = lax.axis_index("c")
        sid = lax.axis_index("s")
        wid = sid * 2 + cid

        zbase = sid * rows_per_tile
        pltpu.sync_copy(zero_hbm.at[pl.ds(zbase, rows_per_tile)],
                        agg_sh.at[pl.ds(zbase, rows_per_tile)])
        if tail_rows:
            @pl.when(sid == 0)
            def _zero_tail():
                pltpu.sync_copy(zero_hbm.at[pl.ds(tail_base, tail_rows)],
                                agg_sh.at[pl.ds(tail_base, tail_rows)])
        plsc.subcore_barrier()

        def load_start(t, u4, b2):
            chunk = t * nw + wid
            base = chunk * _K
            base2 = pl.multiple_of(chunk * (_K // 2), _K // 2)
            pltpu.async_copy(col_hbm.at[pl.ds(base, _K)], col_v.at[u4],
                             sem_col[u4])
            pltpu.async_copy(row_hbm.at[pl.ds(base, _K)], row_v.at[u4],
                             sem_row[u4])
            pltpu.async_copy(ef_hbm.at[pl.ds(base2, _K // 2)], ef_v[b2],
                             sem_ef[b2])

        def gather_start(u4, b2):
            pltpu.make_async_copy(col_hbm.at[pl.ds(0, _K)], col_v.at[u4],
                                  sem_col[u4]).wait()
            pltpu.async_copy(x_hbm.at[col_v.at[u4]], gx_v[b2], sem_gx[b2])

        load_start(0, 0, 0)
        load_start(1, 1, 1)
        gather_start(0, 0)

        def quad_body(q, carry):
            for u in range(4):
                b2 = u % 2
                nb2 = (u + 1) % 2
                t = q * 4 + u
                chunk = t * nw + wid

                @pl.when(chunk + nw < nchunks)
                def _():
                    gather_start((u + 1) % 4, nb2)

                @pl.when(chunk < nchunks)
                def _():
                    pltpu.make_async_copy(ef_hbm.at[pl.ds(0, _K // 2)],
                                          ef_v[b2], sem_ef[b2]).wait()
                    pltpu.make_async_copy(x_hbm.at[col_v.at[u]], gx_v[b2],
                                          sem_gx[b2]).wait()

                    @pl.when(t >= 2)
                    def _():
                        pltpu.make_async_copy(
                            msg_v[b2], agg_sh.at[row_v.at[u]],
                            sem_sc[b2]).wait()

                    @plsc.parallel_loop(0, _K // 2, unroll=4)
                    def _mul(i):
                        for j in range(h // 16):
                            sl = pl.ds(j * 16, 16)
                            w = ef_v[b2][i, sl]
                            ea = lax.bitcast_convert_type(
                                w << 16, jnp.float32)
                            eb = lax.bitcast_convert_type(
                                w & jnp.uint32(0xFFFF0000), jnp.float32)
                            msg_v[b2][2 * i, sl] = ea * gx_v[b2][2 * i, sl]
                            msg_v[b2][2 * i + 1, sl] = (
                                eb * gx_v[b2][2 * i + 1, sl])

                    pltpu.make_async_copy(row_hbm.at[pl.ds(0, _K)],
                                          row_v.at[u], sem_row[u]).wait()
                    pltpu.async_copy(msg_v[b2], agg_sh.at[row_v.at[u]],
                                     sem_sc[b2], add=True)

                    @pl.when(chunk + 2 * nw >= nchunks)
                    def _():
                        pltpu.make_async_copy(
                            msg_v[b2], agg_sh.at[row_v.at[u]],
                            sem_sc[b2]).wait()

                @pl.when(chunk + 2 * nw < nchunks)
                def _():
                    load_start(t + 2, (u + 2) % 4, b2)

            return carry

        lax.fori_loop(0, -(-tpw // 4), quad_body, 0, unroll=False)

        plsc.subcore_barrier()
        pltpu.sync_copy(agg_sh.at[pl.ds(zbase, rows_per_tile)],
                        out_hbm.at[cid, pl.ds(zbase, rows_per_tile)])
        if tail_rows:
            @pl.when(sid == 0)
            def _write_tail():
                pltpu.sync_copy(agg_sh.at[pl.ds(tail_base, tail_rows)],
                                out_hbm.at[cid, pl.ds(tail_base, tail_rows)])

    return sc_kernel(ef, x, row, col, zeros)


def _node_mlp_body(x_ref, p0_ref, p1_ref, w1_ref, b1_ref, w2_ref, b2_ref,
                   out_ref):
    agg = p0_ref[...] + p1_ref[...]
    g = jnp.dot(agg, w1_ref[...], preferred_element_type=jnp.float32)
    g = g + b1_ref[...]
    g = g * jax.nn.sigmoid(g)
    o = jnp.dot(g, w2_ref[...], preferred_element_type=jnp.float32)
    out_ref[...] = x_ref[...] + o + b2_ref[...]


def _node_mlp(x, p0, p1, w1, b1, w2, b2, block_n):
    n, h = x.shape
    grid = n // block_n
    return pl.pallas_call(
        _node_mlp_body,
        grid=(grid,),
        in_specs=[
            pl.BlockSpec((block_n, h), lambda i: (i, 0)),
            pl.BlockSpec((block_n, h), lambda i: (i, 0)),
            pl.BlockSpec((block_n, h), lambda i: (i, 0)),
            pl.BlockSpec((h, h), lambda i: (0, 0)),
            pl.BlockSpec((1, h), lambda i: (0, 0)),
            pl.BlockSpec((h, h), lambda i: (0, 0)),
            pl.BlockSpec((1, h), lambda i: (0, 0)),
        ],
        out_specs=pl.BlockSpec((block_n, h), lambda i: (i, 0)),
        out_shape=jax.ShapeDtypeStruct((n, h), jnp.float32),
    )(x, p0, p1, w1, b1, w2, b2)


def kernel(x, edge_index, edge_rbf, We1, be1, We2, be2, Wn1, bn1, Wn2, bn2):
    n, h = x.shape
    e = edge_rbf.shape[0]
    assert e % _K == 0 and n % 16 == 0 and h % 16 == 0

    row = edge_index[0]
    col = edge_index[1]

    ef = _edge_mlp(edge_rbf, We1, be1.reshape(1, h), We2, be2.reshape(1, h),
                   block_e=3200)

    zeros = jnp.zeros((n, h), jnp.float32)
    partials = _sc_aggregate(ef, x, row, col, zeros)

    return _node_mlp(x, partials[0], partials[1], Wn1, bn1.reshape(1, h),
                     Wn2, bn2.reshape(1, h), block_n=2000)


# speedup vs baseline: 5.6690x; 1.0909x over previous
import functools

import jax
import jax.numpy as jnp
from jax import lax
from jax.experimental import pallas as pl
from jax.experimental.pallas import tpu as pltpu
from jax.experimental.pallas import tpu_sc as plsc


def _edge_mlp_body(rbf_ref, w1_ref, b1_ref, w2_ref, b2_ref, out_ref):
    h = jnp.dot(rbf_ref[...], w1_ref[...], preferred_element_type=jnp.float32)
    h = h + b1_ref[...]
    h = h * jax.nn.sigmoid(h)
    o = jnp.dot(h, w2_ref[...], preferred_element_type=jnp.float32)
    ob = (o + b2_ref[...]).astype(jnp.bfloat16)
    out_ref[...] = pltpu.bitcast(ob, jnp.uint32)


def _edge_mlp(rbf, w1, b1, w2, b2, block_e, start_block, num_blocks):
    e, r = rbf.shape
    h = w1.shape[1]
    return pl.pallas_call(
        _edge_mlp_body,
        grid=(num_blocks,),
        in_specs=[
            pl.BlockSpec((block_e, r), lambda i: (i + start_block, 0)),
            pl.BlockSpec((r, h), lambda i: (0, 0)),
            pl.BlockSpec((1, h), lambda i: (0, 0)),
            pl.BlockSpec((h, h), lambda i: (0, 0)),
            pl.BlockSpec((1, h), lambda i: (0, 0)),
        ],
        out_specs=pl.BlockSpec((block_e // 2, h), lambda i: (i, 0)),
        out_shape=jax.ShapeDtypeStruct((num_blocks * block_e // 2, h),
                                       jnp.uint32),
    )(rbf, w1, b1, w2, b2)


_K = 64


def _sc_aggregate(ef, x, row, col, zeros):
    e2 = ef.shape[0]
    e = e2 * 2
    n, h = x.shape
    nchunks = e // _K
    nw = 32
    tpw = -(-nchunks // nw)
    rows_per_tile = (n // 128) * 8
    tail_base = rows_per_tile * 16
    tail_rows = n - tail_base

    mesh = plsc.VectorSubcoreMesh(core_axis_name="c", subcore_axis_name="s")

    @functools.partial(
        pl.kernel,
        mesh=mesh,
        out_type=jax.ShapeDtypeStruct((2, n, h), jnp.float32),
        scratch_types=[
            pltpu.VMEM((4, _K), jnp.int32),
            pltpu.VMEM((4, _K), jnp.int32),
            pltpu.VMEM((_K, h), jnp.float32),
            pltpu.VMEM((_K, h), jnp.float32),
            pltpu.VMEM((_K // 2, h), jnp.uint32),
            pltpu.VMEM((_K // 2, h), jnp.uint32),
            pltpu.VMEM((_K, h), jnp.float32),
            pltpu.VMEM((_K, h), jnp.float32),
            pltpu.VMEM_SHARED((n, h), jnp.float32),
            [pltpu.SemaphoreType.DMA] * 4,
            [pltpu.SemaphoreType.DMA] * 4,
            [pltpu.SemaphoreType.DMA] * 2,
            [pltpu.SemaphoreType.DMA] * 2,
            [pltpu.SemaphoreType.DMA] * 2,
        ],
    )
    def sc_kernel(ef_hbm, x_hbm, row_hbm, col_hbm, zero_hbm, out_hbm,
                  col_v, row_v, gx0, gx1, ef0, ef1, msg0, msg1, agg_sh,
                  sem_col, sem_row, sem_ef, sem_gx, sem_sc):
        gx_v = (gx0, gx1)
        ef_v = (ef0, ef1)
        msg_v = (msg0, msg1)
        cid = lax.axis_index("c")
        sid = lax.axis_index("s")
        wid = sid * 2 + cid

        zbase = sid * rows_per_tile
        pltpu.sync_copy(zero_hbm.at[pl.ds(zbase, rows_per_tile)],
                        agg_sh.at[pl.ds(zbase, rows_per_tile)])
        if tail_rows:
            @pl.when(sid == 0)
            def _zero_tail():
                pltpu.sync_copy(zero_hbm.at[pl.ds(tail_base, tail_rows)],
                                agg_sh.at[pl.ds(tail_base, tail_rows)])
        plsc.subcore_barrier()

        def load_start(t, u4, b2):
            chunk = t * nw + wid
            base = chunk * _K
            base2 = pl.multiple_of(chunk * (_K // 2), _K // 2)
            pltpu.async_copy(col_hbm.at[pl.ds(base, _K)], col_v.at[u4],
                             sem_col[u4])
            pltpu.async_copy(row_hbm.at[pl.ds(base, _K)], row_v.at[u4],
                             sem_row[u4])
            pltpu.async_copy(ef_hbm.at[pl.ds(base2, _K // 2)], ef_v[b2],
                             sem_ef[b2])

        def gather_start(u4, b2):
            pltpu.make_async_copy(col_hbm.at[pl.ds(0, _K)], col_v.at[u4],
                                  sem_col[u4]).wait()
            pltpu.async_copy(x_hbm.at[col_v.at[u4]], gx_v[b2], sem_gx[b2])

        load_start(0, 0, 0)
        load_start(1, 1, 1)
        gather_start(0, 0)

        def quad_body(q, carry):
            for u in range(4):
                b2 = u % 2
                nb2 = (u + 1) % 2
                t = q * 4 + u
                chunk = t * nw + wid

                @pl.when(chunk + nw < nchunks)
                def _():
                    gather_start((u + 1) % 4, nb2)

                @pl.when(chunk < nchunks)
                def _():
                    pltpu.make_async_copy(ef_hbm.at[pl.ds(0, _K // 2)],
                                          ef_v[b2], sem_ef[b2]).wait()
                    pltpu.make_async_copy(x_hbm.at[col_v.at[u]], gx_v[b2],
                                          sem_gx[b2]).wait()

                    @pl.when(t >= 2)
                    def _():
                        pltpu.make_async_copy(
                            msg_v[b2], agg_sh.at[row_v.at[u]],
                            sem_sc[b2]).wait()

                    @plsc.parallel_loop(0, _K // 2, unroll=4)
                    def _mul(i):
                        for j in range(h // 16):
                            sl = pl.ds(j * 16, 16)
                            w = ef_v[b2][i, sl]
                            ea = lax.bitcast_convert_type(
                                w << 16, jnp.float32)
                            eb = lax.bitcast_convert_type(
                                w & jnp.uint32(0xFFFF0000), jnp.float32)
                            msg_v[b2][2 * i, sl] = ea * gx_v[b2][2 * i, sl]
                            msg_v[b2][2 * i + 1, sl] = (
                                eb * gx_v[b2][2 * i + 1, sl])

                    pltpu.make_async_copy(row_hbm.at[pl.ds(0, _K)],
                                          row_v.at[u], sem_row[u]).wait()
                    pltpu.async_copy(msg_v[b2], agg_sh.at[row_v.at[u]],
                                     sem_sc[b2], add=True)

                    @pl.when(chunk + 2 * nw >= nchunks)
                    def _():
                        pltpu.make_async_copy(
                            msg_v[b2], agg_sh.at[row_v.at[u]],
                            sem_sc[b2]).wait()

                @pl.when(chunk + 2 * nw < nchunks)
                def _():
                    load_start(t + 2, (u + 2) % 4, b2)

            return carry

        lax.fori_loop(0, -(-tpw // 4), quad_body, 0, unroll=False)

        plsc.subcore_barrier()
        pltpu.sync_copy(agg_sh.at[pl.ds(zbase, rows_per_tile)],
                        out_hbm.at[cid, pl.ds(zbase, rows_per_tile)])
        if tail_rows:
            @pl.when(sid == 0)
            def _write_tail():
                pltpu.sync_copy(agg_sh.at[pl.ds(tail_base, tail_rows)],
                                out_hbm.at[cid, pl.ds(tail_base, tail_rows)])

    return sc_kernel(ef, x, row, col, zeros)


def _node_mlp_body(x_ref, p0_ref, p1_ref, p2_ref, p3_ref, w1_ref, b1_ref,
                   w2_ref, b2_ref, out_ref):
    agg = (p0_ref[...] + p1_ref[...]) + (p2_ref[...] + p3_ref[...])
    g = jnp.dot(agg, w1_ref[...], preferred_element_type=jnp.float32)
    g = g + b1_ref[...]
    g = g * jax.nn.sigmoid(g)
    o = jnp.dot(g, w2_ref[...], preferred_element_type=jnp.float32)
    out_ref[...] = x_ref[...] + o + b2_ref[...]


def _node_mlp(x, p0, p1, p2, p3, w1, b1, w2, b2, block_n):
    n, h = x.shape
    grid = n // block_n
    return pl.pallas_call(
        _node_mlp_body,
        grid=(grid,),
        in_specs=[
            pl.BlockSpec((block_n, h), lambda i: (i, 0)),
            pl.BlockSpec((block_n, h), lambda i: (i, 0)),
            pl.BlockSpec((block_n, h), lambda i: (i, 0)),
            pl.BlockSpec((block_n, h), lambda i: (i, 0)),
            pl.BlockSpec((block_n, h), lambda i: (i, 0)),
            pl.BlockSpec((h, h), lambda i: (0, 0)),
            pl.BlockSpec((1, h), lambda i: (0, 0)),
            pl.BlockSpec((h, h), lambda i: (0, 0)),
            pl.BlockSpec((1, h), lambda i: (0, 0)),
        ],
        out_specs=pl.BlockSpec((block_n, h), lambda i: (i, 0)),
        out_shape=jax.ShapeDtypeStruct((n, h), jnp.float32),
    )(x, p0, p1, p2, p3, w1, b1, w2, b2)


def kernel(x, edge_index, edge_rbf, We1, be1, We2, be2, Wn1, bn1, Wn2, bn2):
    n, h = x.shape
    e = edge_rbf.shape[0]
    assert e % _K == 0 and n % 16 == 0 and h % 16 == 0

    row = edge_index[0]
    col = edge_index[1]

    block_e = 3200
    nb = e // block_e
    eh = (nb // 2) * block_e
    b1r = be1.reshape(1, h)
    b2r = be2.reshape(1, h)
    ef_a = _edge_mlp(edge_rbf, We1, b1r, We2, b2r, block_e=block_e,
                     start_block=0, num_blocks=nb // 2)
    ef_b = _edge_mlp(edge_rbf, We1, b1r, We2, b2r, block_e=block_e,
                     start_block=nb // 2, num_blocks=nb - nb // 2)
    zeros = jnp.zeros((n, h), jnp.float32)
    pa = _sc_aggregate(ef_a, x, row[:eh], col[:eh], zeros)
    pb = _sc_aggregate(ef_b, x, row[eh:], col[eh:], zeros)

    return _node_mlp(x, pa[0], pa[1], pb[0], pb[1], Wn1, bn1.reshape(1, h),
                     Wn2, bn2.reshape(1, h), block_n=2000)
